# baseline (speedup 1.0000x reference)
import functools

import jax
import jax.numpy as jnp
from jax import lax
from jax.experimental import pallas as pl
from jax.experimental.pallas import tpu as pltpu
from jax.experimental.pallas import tpu_sc as plsc

N = 10000
E = 320000
D = 128
K = 10
ALPHA = 0.1
BETA = 0.9
GAMMA = BETA ** K + ALPHA * sum(BETA ** i for i in range(K))

NC = 2
NS = 16
EPS = E // NS
LB = 5120
BK = 320
TR = BK
ALOC = 328
BCAP = 896
BSTR = 1040
BB = 128
NBB = NS * BCAP // BB
NBUF = 2
NDP = 10240

_mesh = plsc.VectorSubcoreMesh(core_axis_name="c", subcore_axis_name="s")
_params = pltpu.CompilerParams(needs_layout_passes=False)


@functools.partial(
    pl.kernel,
    out_type=[
        jax.ShapeDtypeStruct((NC * NS * NS * BCAP,), jnp.int32),
        jax.ShapeDtypeStruct((NC * NS * NS * BCAP,), jnp.int32),
        jax.ShapeDtypeStruct((NS, NDP), jnp.float32),
    ],
    mesh=_mesh,
    compiler_params=_params,
    scratch_types=[
        pltpu.VMEM((EPS,), jnp.int32),
        pltpu.VMEM((EPS,), jnp.int32),
        pltpu.VMEM((NS * BSTR,), jnp.int32),
        pltpu.VMEM((NS * BSTR,), jnp.int32),
        pltpu.VMEM((NDP,), jnp.float32),
    ],
)
def _sc_partition(row_hbm, col_hbm, brow_o, bcol_o, deg_o,
                  rowv, colv, brow, bcol, degl):
    c = lax.axis_index("c")
    s2 = lax.axis_index("s")
    pltpu.sync_copy(row_hbm.at[pl.ds(s2 * EPS, EPS)], rowv)
    pltpu.sync_copy(col_hbm.at[pl.ds(s2 * EPS, EPS)], colv)

    trash16 = jnp.full((16,), TR, jnp.int32)
    zero16i = jnp.zeros((16,), jnp.int32)

    def _fill(i, _):
        brow[pl.ds(i * 16, 16)] = trash16
        bcol[pl.ds(i * 16, 16)] = zero16i
        return 0

    lax.fori_loop(0, NS * BSTR // 16, _fill, 0)

    @pl.when(c == 0)
    def _deg():
        def _zdeg(i, _):
            degl[pl.ds(i * 16, 16)] = jnp.zeros((16,), jnp.float32)
            return 0

        lax.fori_loop(0, NDP // 16, _zdeg, 0)
        ones = jnp.ones((16,), jnp.float32)

        def _dscan(i, _):
            plsc.addupdate_scatter(degl, [rowv[pl.ds(i * 16, 16)]], ones)
            return 0

        lax.fori_loop(0, EPS // 16, _dscan, 0)
        pltpu.sync_copy(degl, deg_o.at[s2])

    base_bk = c * NS

    def _scan(i, offs):
        r = rowv[pl.ds(i * 16, 16)]
        cc = colv[pl.ds(i * 16, 16)]
        bk = r // BK
        rl = r % BK
        new_offs = []
        for k in range(NS):
            m = bk == (base_bk + k)
            off = offs[k]
            plsc.store_compressed(brow.at[pl.ds(k * BSTR + off, 16)], rl, mask=m)
            plsc.store_compressed(bcol.at[pl.ds(k * BSTR + off, 16)], cc, mask=m)
            new_offs.append(off + jnp.sum(m.astype(jnp.int32)))
        return tuple(new_offs)

    lax.fori_loop(0, EPS // 16, _scan, tuple(jnp.int32(0) for _ in range(NS)))

    for k in range(NS):
        fi = (c * NS + k) * NS + s2
        pltpu.sync_copy(brow.at[pl.ds(k * BSTR, BCAP)], brow_o.at[pl.ds(fi * BCAP, BCAP)])
        pltpu.sync_copy(bcol.at[pl.ds(k * BSTR, BCAP)], bcol_o.at[pl.ds(fi * BCAP, BCAP)])


@functools.partial(
    pl.kernel,
    out_type=jax.ShapeDtypeStruct((NC, LB, D), jnp.float32),
    mesh=_mesh,
    compiler_params=_params,
    scratch_types=[
        pltpu.VMEM((NBB, BB), jnp.int32),
        pltpu.VMEM((NBB, BB), jnp.int32),
        pltpu.VMEM((NBUF, BB, D), jnp.float32),
        pltpu.VMEM((ALOC, D), jnp.float32),
        pltpu.SemaphoreType.DMA((NBUF,)),
    ],
)
def _sc_spmm(ot_hbm, brow_hbm, bcol_hbm, acc_out,
             browv, bcolv, gbufs, accl, gsem):
    c = lax.axis_index("c")
    t = lax.axis_index("s")

    pltpu.sync_copy(brow_hbm.at[c, t], browv)
    pltpu.sync_copy(bcol_hbm.at[c, t], bcolv)

    z16 = jnp.zeros((16,), jnp.float32)

    def _zrow(i, _):
        for j in range(D // 16):
            accl[i, pl.ds(j * 16, 16)] = z16
        return 0

    lax.fori_loop(0, ALOC, _zrow, 0)

    for j in range(NBUF):
        pltpu.async_copy(ot_hbm.at[bcolv.at[j]], gbufs.at[j], gsem.at[j])

    def _step(g, _):
        for j in range(NBUF):
            b = g * NBUF + j
            pltpu.make_async_copy(ot_hbm.at[bcolv.at[b]], gbufs.at[j],
                                  gsem.at[j]).wait()

            def _rows(i, _):
                dst16 = browv[b, pl.ds(i * 16, 16)]
                for u in range(16):
                    r = i * 16 + u
                    dst = dst16[u]
                    for q in range(D // 16):
                        sl = pl.ds(q * 16, 16)
                        plsc.addupdate(accl.at[dst, sl], gbufs[j, r, sl])
                return 0

            lax.fori_loop(0, BB // 16, _rows, 0)
            nb = b + NBUF

            @pl.when(nb < NBB)
            def _refill():
                pltpu.async_copy(ot_hbm.at[bcolv.at[nb]], gbufs.at[j], gsem.at[j])

        return 0

    lax.fori_loop(0, NBB // NBUF, _step, 0)

    pltpu.sync_copy(accl.at[pl.ds(0, BK)], acc_out.at[c, pl.ds(t * BK, BK)])


def _scales_body(dp_ref, s_ref, b2_ref, bg_ref):
    d = jnp.sum(dp_ref[...], axis=0, keepdims=True)[:, :N]
    s = jnp.where(d > 0.0, lax.rsqrt(jnp.maximum(d, 1e-30)), 0.0)
    s_ref[...] = s
    b2_ref[...] = BETA * s * s
    bg_ref[...] = (BETA / GAMMA) * s


def _tc_scales(deg_parts):
    return pl.pallas_call(
        _scales_body,
        out_shape=[jax.ShapeDtypeStruct((1, N), jnp.float32)] * 3,
    )(deg_parts)


_BR = 2000


def _rowscale_body(s_ref, h_ref, o_ref):
    o_ref[...] = s_ref[...] * h_ref[...]


def _tc_rowscale(s_col, h):
    return pl.pallas_call(
        _rowscale_body,
        grid=(N // _BR,),
        in_specs=[
            pl.BlockSpec((_BR, 1), lambda i: (i, 0)),
            pl.BlockSpec((_BR, D), lambda i: (i, 0)),
        ],
        out_specs=pl.BlockSpec((_BR, D), lambda i: (i, 0)),
        out_shape=jax.ShapeDtypeStruct((N, D), jnp.float32),
    )(s_col, h)


def _combine_body(addmul, acc_ref, sc_ref, add_ref, o_ref):
    o_ref[...] = sc_ref[...] * acc_ref[...] + addmul * add_ref[...]


def _tc_combine(acc_flat, scale_col, addsrc, addmul):
    return pl.pallas_call(
        functools.partial(_combine_body, addmul),
        grid=(N // _BR,),
        in_specs=[
            pl.BlockSpec((_BR, D), lambda i: (i, 0)),
            pl.BlockSpec((_BR, 1), lambda i: (i, 0)),
            pl.BlockSpec((_BR, D), lambda i: (i, 0)),
        ],
        out_specs=pl.BlockSpec((_BR, D), lambda i: (i, 0)),
        out_shape=jax.ShapeDtypeStruct((N, D), jnp.float32),
    )(acc_flat, scale_col, addsrc)


def kernel(x, edge_index):
    row = edge_index[0]
    col = edge_index[1]

    brow, bcol, deg_parts = _sc_partition(row, col)
    browr = brow.reshape(NC, NS, NBB, BB)
    bcolr = bcol.reshape(NC, NS, NBB, BB)

    s_row, b2_row, bg_row = _tc_scales(deg_parts)
    s_col = s_row.reshape(N, 1)
    b2 = b2_row.reshape(N, 1)
    bg = bg_row.reshape(N, 1)

    ot = _tc_rowscale(s_col, x)
    ot0 = ot
    for k in range(K):
        acc = _sc_spmm(ot, browr, bcolr)
        acc_flat = acc.reshape(NC * LB, D)
        if k < K - 1:
            ot = _tc_combine(acc_flat, b2, ot0, ALPHA)
        else:
            out = _tc_combine(acc_flat, bg, x, ALPHA / GAMMA)
    return out


# speedup vs baseline: 1.0054x; 1.0054x over previous
import functools

import jax
import jax.numpy as jnp
from jax import lax
from jax.experimental import pallas as pl
from jax.experimental.pallas import tpu as pltpu
from jax.experimental.pallas import tpu_sc as plsc

N = 10000
E = 320000
D = 128
K = 10
ALPHA = 0.1
BETA = 0.9
GAMMA = BETA ** K + ALPHA * sum(BETA ** i for i in range(K))

NC = 2
NS = 16
EPS = E // NS
LB = 5120
BK = 320
TR = BK
ALOC = 328
BCAP = 896
BSTR = 1040
BB = 128
NBB = NS * BCAP // BB
NBUF = 2
NDP = 10240

_mesh = plsc.VectorSubcoreMesh(core_axis_name="c", subcore_axis_name="s")
_params = pltpu.CompilerParams(needs_layout_passes=False)


@functools.partial(
    pl.kernel,
    out_type=[
        jax.ShapeDtypeStruct((NC * NS * NS * BCAP,), jnp.int32),
        jax.ShapeDtypeStruct((NC * NS * NS * BCAP,), jnp.int32),
        jax.ShapeDtypeStruct((NS, NDP), jnp.float32),
    ],
    mesh=_mesh,
    compiler_params=_params,
    scratch_types=[
        pltpu.VMEM((EPS,), jnp.int32),
        pltpu.VMEM((EPS,), jnp.int32),
        pltpu.VMEM((NS * BSTR,), jnp.int32),
        pltpu.VMEM((NS * BSTR,), jnp.int32),
        pltpu.VMEM((NDP,), jnp.float32),
    ],
)
def _sc_partition(row_hbm, col_hbm, brow_o, bcol_o, deg_o,
                  rowv, colv, brow, bcol, degl):
    c = lax.axis_index("c")
    s2 = lax.axis_index("s")
    pltpu.sync_copy(row_hbm.at[pl.ds(s2 * EPS, EPS)], rowv)
    pltpu.sync_copy(col_hbm.at[pl.ds(s2 * EPS, EPS)], colv)

    trash16 = jnp.full((16,), TR, jnp.int32)
    zero16i = jnp.zeros((16,), jnp.int32)

    def _fill(i, _):
        brow[pl.ds(i * 16, 16)] = trash16
        bcol[pl.ds(i * 16, 16)] = zero16i
        return 0

    lax.fori_loop(0, NS * BSTR // 16, _fill, 0)

    @pl.when(c == 0)
    def _deg():
        def _zdeg(i, _):
            degl[pl.ds(i * 16, 16)] = jnp.zeros((16,), jnp.float32)
            return 0

        lax.fori_loop(0, NDP // 16, _zdeg, 0)
        ones = jnp.ones((16,), jnp.float32)

        def _dscan(i, _):
            plsc.addupdate_scatter(degl, [rowv[pl.ds(i * 16, 16)]], ones)
            return 0

        lax.fori_loop(0, EPS // 16, _dscan, 0)
        pltpu.sync_copy(degl, deg_o.at[s2])

    base_bk = c * NS

    def _scan(i, offs):
        r = rowv[pl.ds(i * 16, 16)]
        cc = colv[pl.ds(i * 16, 16)]
        bk = r // BK
        rl = r % BK
        new_offs = []
        for k in range(NS):
            m = bk == (base_bk + k)
            off = offs[k]
            plsc.store_compressed(brow.at[pl.ds(k * BSTR + off, 16)], rl, mask=m)
            plsc.store_compressed(bcol.at[pl.ds(k * BSTR + off, 16)], cc, mask=m)
            new_offs.append(off + jnp.sum(m.astype(jnp.int32)))
        return tuple(new_offs)

    lax.fori_loop(0, EPS // 16, _scan, tuple(jnp.int32(0) for _ in range(NS)))

    for k in range(NS):
        fi = (c * NS + k) * NS + s2
        pltpu.sync_copy(brow.at[pl.ds(k * BSTR, BCAP)], brow_o.at[pl.ds(fi * BCAP, BCAP)])
        pltpu.sync_copy(bcol.at[pl.ds(k * BSTR, BCAP)], bcol_o.at[pl.ds(fi * BCAP, BCAP)])


@functools.partial(
    pl.kernel,
    out_type=jax.ShapeDtypeStruct((NC, LB, D), jnp.float32),
    mesh=_mesh,
    compiler_params=_params,
    scratch_types=[
        pltpu.VMEM((NBB, BB), jnp.int32),
        pltpu.VMEM((NBB, BB), jnp.int32),
        pltpu.VMEM((NBUF, BB, D), jnp.float32),
        pltpu.VMEM((ALOC, D), jnp.float32),
        pltpu.SemaphoreType.DMA((NBUF,)),
    ],
)
def _sc_spmm(ot_hbm, brow_hbm, bcol_hbm, acc_out,
             browv, bcolv, gbufs, accl, gsem):
    c = lax.axis_index("c")
    t = lax.axis_index("s")

    pltpu.sync_copy(brow_hbm.at[c, t], browv)
    pltpu.sync_copy(bcol_hbm.at[c, t], bcolv)

    z16 = jnp.zeros((16,), jnp.float32)

    def _zrow(i, _):
        for j in range(D // 16):
            accl[i, pl.ds(j * 16, 16)] = z16
        return 0

    lax.fori_loop(0, ALOC, _zrow, 0)

    for j in range(NBUF):
        pltpu.async_copy(ot_hbm.at[bcolv.at[j]], gbufs.at[j], gsem.at[j])

    def _step(g, _):
        for j in range(NBUF):
            b = g * NBUF + j
            pltpu.make_async_copy(ot_hbm.at[bcolv.at[b]], gbufs.at[j],
                                  gsem.at[j]).wait()

            nb = b + NBUF

            @pl.when(nb < NBB)
            def _refill():
                pltpu.async_copy(ot_hbm.at[bcolv.at[nb]], gbufs.at[j], gsem.at[j])

        return 0

    lax.fori_loop(0, NBB // NBUF, _step, 0)

    pltpu.sync_copy(accl.at[pl.ds(0, BK)], acc_out.at[c, pl.ds(t * BK, BK)])


def _scales_body(dp_ref, s_ref, b2_ref, bg_ref):
    d = jnp.sum(dp_ref[...], axis=0, keepdims=True)[:, :N]
    s = jnp.where(d > 0.0, lax.rsqrt(jnp.maximum(d, 1e-30)), 0.0)
    s_ref[...] = s
    b2_ref[...] = BETA * s * s
    bg_ref[...] = (BETA / GAMMA) * s


def _tc_scales(deg_parts):
    return pl.pallas_call(
        _scales_body,
        out_shape=[jax.ShapeDtypeStruct((1, N), jnp.float32)] * 3,
    )(deg_parts)


_BR = 2000


def _rowscale_body(s_ref, h_ref, o_ref):
    o_ref[...] = s_ref[...] * h_ref[...]


def _tc_rowscale(s_col, h):
    return pl.pallas_call(
        _rowscale_body,
        grid=(N // _BR,),
        in_specs=[
            pl.BlockSpec((_BR, 1), lambda i: (i, 0)),
            pl.BlockSpec((_BR, D), lambda i: (i, 0)),
        ],
        out_specs=pl.BlockSpec((_BR, D), lambda i: (i, 0)),
        out_shape=jax.ShapeDtypeStruct((N, D), jnp.float32),
    )(s_col, h)


def _combine_body(addmul, acc_ref, sc_ref, add_ref, o_ref):
    o_ref[...] = sc_ref[...] * acc_ref[...] + addmul * add_ref[...]


def _tc_combine(acc_flat, scale_col, addsrc, addmul):
    return pl.pallas_call(
        functools.partial(_combine_body, addmul),
        grid=(N // _BR,),
        in_specs=[
            pl.BlockSpec((_BR, D), lambda i: (i, 0)),
            pl.BlockSpec((_BR, 1), lambda i: (i, 0)),
            pl.BlockSpec((_BR, D), lambda i: (i, 0)),
        ],
        out_specs=pl.BlockSpec((_BR, D), lambda i: (i, 0)),
        out_shape=jax.ShapeDtypeStruct((N, D), jnp.float32),
    )(acc_flat, scale_col, addsrc)


def kernel(x, edge_index):
    row = edge_index[0]
    col = edge_index[1]

    brow, bcol, deg_parts = _sc_partition(row, col)
    browr = brow.reshape(NC, NS, NBB, BB)
    bcolr = bcol.reshape(NC, NS, NBB, BB)

    s_row, b2_row, bg_row = _tc_scales(deg_parts)
    s_col = s_row.reshape(N, 1)
    b2 = b2_row.reshape(N, 1)
    bg = bg_row.reshape(N, 1)

    ot = _tc_rowscale(s_col, x)
    ot0 = ot
    for k in range(K):
        acc = _sc_spmm(ot, browr, bcolr)
        acc_flat = acc.reshape(NC * LB, D)
        if k < K - 1:
            ot = _tc_combine(acc_flat, b2, ot0, ALPHA)
        else:
            out = _tc_combine(acc_flat, bg, x, ALPHA / GAMMA)
    return out


# speedup vs baseline: 12.5456x; 12.4778x over previous
import functools

import jax
import jax.numpy as jnp
from jax import lax
from jax.experimental import pallas as pl
from jax.experimental.pallas import tpu as pltpu
from jax.experimental.pallas import tpu_sc as plsc

N = 10000
E = 320000
D = 128
K = 10
ALPHA = 0.1
BETA = 0.9
GAMMA = BETA ** K + ALPHA * sum(BETA ** i for i in range(K))

NC = 2
NS = 16
EPS = E // NS
LB = 5120
BK = 320
TR = BK
ALOC = 328
BCAP = 896
BSTR = 1040
BB = 128
NBB = NS * BCAP // BB
NBUF = 2
NDP = 10240

_mesh = plsc.VectorSubcoreMesh(core_axis_name="c", subcore_axis_name="s")
_params = pltpu.CompilerParams(needs_layout_passes=False)


@functools.partial(
    pl.kernel,
    out_type=[
        jax.ShapeDtypeStruct((NC * NS * NS * BCAP,), jnp.int32),
        jax.ShapeDtypeStruct((NC * NS * NS * BCAP,), jnp.int32),
        jax.ShapeDtypeStruct((NS, NDP), jnp.float32),
    ],
    mesh=_mesh,
    compiler_params=_params,
    scratch_types=[
        pltpu.VMEM((EPS,), jnp.int32),
        pltpu.VMEM((EPS,), jnp.int32),
        pltpu.VMEM((NS * BSTR,), jnp.int32),
        pltpu.VMEM((NS * BSTR,), jnp.int32),
        pltpu.VMEM((NDP,), jnp.float32),
    ],
)
def _sc_partition(row_hbm, col_hbm, brow_o, bcol_o, deg_o,
                  rowv, colv, brow, bcol, degl):
    c = lax.axis_index("c")
    s2 = lax.axis_index("s")
    pltpu.sync_copy(row_hbm.at[pl.ds(s2 * EPS, EPS)], rowv)
    pltpu.sync_copy(col_hbm.at[pl.ds(s2 * EPS, EPS)], colv)

    trash16 = jnp.full((16,), TR, jnp.int32)
    iota16 = lax.iota(jnp.int32, 16)

    def _fill(i, _):
        brow[pl.ds(i * 16, 16)] = trash16
        bcol[pl.ds(i * 16, 16)] = jnp.bitwise_and(i * 16 + iota16, 8191)
        return 0

    lax.fori_loop(0, NS * BSTR // 16, _fill, 0)

    @pl.when(c == 0)
    def _deg():
        def _zdeg(i, _):
            degl[pl.ds(i * 16, 16)] = jnp.zeros((16,), jnp.float32)
            return 0

        lax.fori_loop(0, NDP // 16, _zdeg, 0)
        ones = jnp.ones((16,), jnp.float32)

        def _dscan(i, _):
            plsc.addupdate_scatter(degl, [rowv[pl.ds(i * 16, 16)]], ones)
            return 0

        lax.fori_loop(0, EPS // 16, _dscan, 0)
        pltpu.sync_copy(degl, deg_o.at[s2])

    base_bk = c * NS

    def _scan(i, offs):
        r = rowv[pl.ds(i * 16, 16)]
        cc = colv[pl.ds(i * 16, 16)]
        bk = r // BK
        rl = r % BK
        new_offs = []
        for k in range(NS):
            m = bk == (base_bk + k)
            off = offs[k]
            plsc.store_compressed(brow.at[pl.ds(k * BSTR + off, 16)], rl, mask=m)
            plsc.store_compressed(bcol.at[pl.ds(k * BSTR + off, 16)], cc, mask=m)
            new_offs.append(off + jnp.sum(m.astype(jnp.int32)))
        return tuple(new_offs)

    lax.fori_loop(0, EPS // 16, _scan, tuple(jnp.int32(0) for _ in range(NS)))

    for k in range(NS):
        fi = (c * NS + k) * NS + s2
        pltpu.sync_copy(brow.at[pl.ds(k * BSTR, BCAP)], brow_o.at[pl.ds(fi * BCAP, BCAP)])
        pltpu.sync_copy(bcol.at[pl.ds(k * BSTR, BCAP)], bcol_o.at[pl.ds(fi * BCAP, BCAP)])


@functools.partial(
    pl.kernel,
    out_type=jax.ShapeDtypeStruct((NC, LB, D), jnp.float32),
    mesh=_mesh,
    compiler_params=_params,
    scratch_types=[
        pltpu.VMEM((NBB, BB), jnp.int32),
        pltpu.VMEM((NBB, BB), jnp.int32),
        pltpu.VMEM((NBUF, BB, D), jnp.float32),
        pltpu.VMEM((ALOC, D), jnp.float32),
        pltpu.SemaphoreType.DMA((NBUF,)),
    ],
)
def _sc_spmm(ot_hbm, brow_hbm, bcol_hbm, acc_out,
             browv, bcolv, gbufs, accl, gsem):
    c = lax.axis_index("c")
    t = lax.axis_index("s")

    pltpu.sync_copy(brow_hbm.at[c, t], browv)
    pltpu.sync_copy(bcol_hbm.at[c, t], bcolv)

    z16 = jnp.zeros((16,), jnp.float32)

    def _zrow(i, _):
        for j in range(D // 16):
            accl[i, pl.ds(j * 16, 16)] = z16
        return 0

    lax.fori_loop(0, ALOC, _zrow, 0)

    for j in range(NBUF):
        pltpu.async_copy(ot_hbm.at[bcolv.at[j]], gbufs.at[j], gsem.at[j])

    def _step(g, _):
        for j in range(NBUF):
            b = g * NBUF + j
            pltpu.make_async_copy(ot_hbm.at[bcolv.at[b]], gbufs.at[j],
                                  gsem.at[j]).wait()

            def _rows(i, _):
                dst16 = browv[b, pl.ds(i * 16, 16)]
                for u in range(16):
                    r = i * 16 + u
                    dst = dst16[u]
                    for q in range(D // 16):
                        sl = pl.ds(q * 16, 16)
                        plsc.addupdate(accl.at[dst, sl], gbufs[j, r, sl])
                return 0

            lax.fori_loop(0, BB // 16, _rows, 0)
            nb = b + NBUF

            @pl.when(nb < NBB)
            def _refill():
                pltpu.async_copy(ot_hbm.at[bcolv.at[nb]], gbufs.at[j], gsem.at[j])

        return 0

    lax.fori_loop(0, NBB // NBUF, _step, 0)

    pltpu.sync_copy(accl.at[pl.ds(0, BK)], acc_out.at[c, pl.ds(t * BK, BK)])


def _scales_body(dp_ref, s_ref, b2_ref, bg_ref):
    d = jnp.sum(dp_ref[...], axis=0, keepdims=True)[:, :N]
    s = jnp.where(d > 0.0, lax.rsqrt(jnp.maximum(d, 1e-30)), 0.0)
    s_ref[...] = s
    b2_ref[...] = BETA * s * s
    bg_ref[...] = (BETA / GAMMA) * s


def _tc_scales(deg_parts):
    return pl.pallas_call(
        _scales_body,
        out_shape=[jax.ShapeDtypeStruct((1, N), jnp.float32)] * 3,
    )(deg_parts)


_BR = 2000


def _rowscale_body(s_ref, h_ref, o_ref):
    o_ref[...] = s_ref[...] * h_ref[...]


def _tc_rowscale(s_col, h):
    return pl.pallas_call(
        _rowscale_body,
        grid=(N // _BR,),
        in_specs=[
            pl.BlockSpec((_BR, 1), lambda i: (i, 0)),
            pl.BlockSpec((_BR, D), lambda i: (i, 0)),
        ],
        out_specs=pl.BlockSpec((_BR, D), lambda i: (i, 0)),
        out_shape=jax.ShapeDtypeStruct((N, D), jnp.float32),
    )(s_col, h)


def _combine_body(addmul, acc_ref, sc_ref, add_ref, o_ref):
    o_ref[...] = sc_ref[...] * acc_ref[...] + addmul * add_ref[...]


def _tc_combine(acc_flat, scale_col, addsrc, addmul):
    return pl.pallas_call(
        functools.partial(_combine_body, addmul),
        grid=(N // _BR,),
        in_specs=[
            pl.BlockSpec((_BR, D), lambda i: (i, 0)),
            pl.BlockSpec((_BR, 1), lambda i: (i, 0)),
            pl.BlockSpec((_BR, D), lambda i: (i, 0)),
        ],
        out_specs=pl.BlockSpec((_BR, D), lambda i: (i, 0)),
        out_shape=jax.ShapeDtypeStruct((N, D), jnp.float32),
    )(acc_flat, scale_col, addsrc)


def kernel(x, edge_index):
    row = edge_index[0]
    col = edge_index[1]

    brow, bcol, deg_parts = _sc_partition(row, col)
    browr = brow.reshape(NC, NS, NBB, BB)
    bcolr = bcol.reshape(NC, NS, NBB, BB)

    s_row, b2_row, bg_row = _tc_scales(deg_parts)
    s_col = s_row.reshape(N, 1)
    b2 = b2_row.reshape(N, 1)
    bg = bg_row.reshape(N, 1)

    ot = _tc_rowscale(s_col, x)
    ot0 = ot
    for k in range(K):
        acc = _sc_spmm(ot, browr, bcolr)
        acc_flat = acc.reshape(NC * LB, D)
        if k < K - 1:
            ot = _tc_combine(acc_flat, b2, ot0, ALPHA)
        else:
            out = _tc_combine(acc_flat, bg, x, ALPHA / GAMMA)
    return out


# speedup vs baseline: 19.8821x; 1.5848x over previous
import functools

import jax
import jax.numpy as jnp
from jax import lax
from jax.experimental import pallas as pl
from jax.experimental.pallas import tpu as pltpu
from jax.experimental.pallas import tpu_sc as plsc

N = 10000
E = 320000
D = 128
K = 10
ALPHA = 0.1
BETA = 0.9
GAMMA = BETA ** K + ALPHA * sum(BETA ** i for i in range(K))

NC = 2
NS = 16
EPS = E // NS
LB = 5120
BK = 320
TR = BK
ALOC = 328
BCAP = 896
BSTR = 1040
BB = 128
NBB = NS * BCAP // BB
NBUF = 2
NDP = 10240

_mesh = plsc.VectorSubcoreMesh(core_axis_name="c", subcore_axis_name="s")
_params = pltpu.CompilerParams(needs_layout_passes=False)


@functools.partial(
    pl.kernel,
    out_type=[
        jax.ShapeDtypeStruct((NC * NS * NS * BCAP,), jnp.int32),
        jax.ShapeDtypeStruct((NC * NS * NS * BCAP,), jnp.int32),
        jax.ShapeDtypeStruct((NS, NDP), jnp.float32),
    ],
    mesh=_mesh,
    compiler_params=_params,
    scratch_types=[
        pltpu.VMEM((EPS,), jnp.int32),
        pltpu.VMEM((EPS,), jnp.int32),
        pltpu.VMEM((NS * BSTR,), jnp.int32),
        pltpu.VMEM((NS * BSTR,), jnp.int32),
        pltpu.VMEM((NDP,), jnp.float32),
    ],
)
def _sc_partition(row_hbm, col_hbm, brow_o, bcol_o, deg_o,
                  rowv, colv, brow, bcol, degl):
    c = lax.axis_index("c")
    s2 = lax.axis_index("s")
    pltpu.sync_copy(row_hbm.at[pl.ds(s2 * EPS, EPS)], rowv)
    pltpu.sync_copy(col_hbm.at[pl.ds(s2 * EPS, EPS)], colv)

    trash16 = jnp.full((16,), TR, jnp.int32)
    iota16 = lax.iota(jnp.int32, 16)

    def _fill(i, _):
        brow[pl.ds(i * 16, 16)] = trash16
        bcol[pl.ds(i * 16, 16)] = jnp.bitwise_and(i * 16 + iota16, 8191)
        return 0

    lax.fori_loop(0, NS * BSTR // 16, _fill, 0)

    @pl.when(c == 0)
    def _deg():
        def _zdeg(i, _):
            degl[pl.ds(i * 16, 16)] = jnp.zeros((16,), jnp.float32)
            return 0

        lax.fori_loop(0, NDP // 16, _zdeg, 0)
        ones = jnp.ones((16,), jnp.float32)

        def _dscan(i, _):
            plsc.addupdate_scatter(degl, [rowv[pl.ds(i * 16, 16)]], ones)
            return 0

        lax.fori_loop(0, EPS // 16, _dscan, 0)
        pltpu.sync_copy(degl, deg_o.at[s2])

    base_bk = c * NS

    def _scan(i, offs):
        r = rowv[pl.ds(i * 16, 16)]
        cc = colv[pl.ds(i * 16, 16)]
        bk = r // BK
        rl = r % BK
        new_offs = []
        for k in range(NS):
            m = bk == (base_bk + k)
            off = offs[k]
            plsc.store_compressed(brow.at[pl.ds(k * BSTR + off, 16)], rl, mask=m)
            plsc.store_compressed(bcol.at[pl.ds(k * BSTR + off, 16)], cc, mask=m)
            new_offs.append(off + jnp.sum(m.astype(jnp.int32)))
        return tuple(new_offs)

    lax.fori_loop(0, EPS // 16, _scan, tuple(jnp.int32(0) for _ in range(NS)))

    for k in range(NS):
        fi = (c * NS + k) * NS + s2
        pltpu.sync_copy(brow.at[pl.ds(k * BSTR, BCAP)], brow_o.at[pl.ds(fi * BCAP, BCAP)])
        pltpu.sync_copy(bcol.at[pl.ds(k * BSTR, BCAP)], bcol_o.at[pl.ds(fi * BCAP, BCAP)])


@functools.partial(
    pl.kernel,
    out_type=jax.ShapeDtypeStruct((NC, LB, D), jnp.float32),
    mesh=_mesh,
    compiler_params=_params,
    scratch_types=[
        pltpu.VMEM((NBB, BB), jnp.int32),
        pltpu.VMEM((NBB, BB), jnp.int32),
        pltpu.VMEM((NBUF, BB, D), jnp.float32),
        pltpu.VMEM((ALOC, D), jnp.float32),
        pltpu.SemaphoreType.DMA((NBUF,)),
    ],
)
def _sc_spmm(ot_hbm, brow_hbm, bcol_hbm, acc_out,
             browv, bcolv, gbufs, accl, gsem):
    c = lax.axis_index("c")
    t = lax.axis_index("s")

    pltpu.sync_copy(brow_hbm.at[c, t], browv)
    pltpu.sync_copy(bcol_hbm.at[c, t], bcolv)

    z16 = jnp.zeros((16,), jnp.float32)

    def _zrow(i, _):
        for j in range(D // 16):
            accl[i, pl.ds(j * 16, 16)] = z16
        return 0

    lax.fori_loop(0, ALOC, _zrow, 0)

    for j in range(NBUF):
        pltpu.async_copy(ot_hbm.at[bcolv.at[j]], gbufs.at[j], gsem.at[j])

    def _step(g, _):
        for j in range(NBUF):
            b = g * NBUF + j
            pltpu.make_async_copy(ot_hbm.at[bcolv.at[b]], gbufs.at[j],
                                  gsem.at[j]).wait()

            def _rows(i, _):
                dst16 = browv[b, pl.ds(i * 16, 16)]
                for u in range(0, 16, 2):
                    r0 = i * 16 + u
                    r1 = r0 + 1
                    d0 = dst16[u]
                    d1 = dst16[u + 1]
                    for q in range(D // 16):
                        sl = pl.ds(q * 16, 16)
                        v0 = gbufs[j, r0, sl]
                        v1 = gbufs[j, r1, sl]
                        plsc.addupdate(accl.at[d0, sl], v0)
                        plsc.addupdate(accl.at[d1, sl], v1)
                return 0

            lax.fori_loop(0, BB // 16, _rows, 0)
            nb = b + NBUF

            @pl.when(nb < NBB)
            def _refill():
                pltpu.async_copy(ot_hbm.at[bcolv.at[nb]], gbufs.at[j], gsem.at[j])

        return 0

    lax.fori_loop(0, NBB // NBUF, _step, 0)

    pltpu.sync_copy(accl.at[pl.ds(0, BK)], acc_out.at[c, pl.ds(t * BK, BK)])


def _scales_body(dp_ref, s_ref, b2_ref, bg_ref):
    d = jnp.sum(dp_ref[...], axis=0, keepdims=True)[:, :N]
    s = jnp.where(d > 0.0, lax.rsqrt(jnp.maximum(d, 1e-30)), 0.0)
    s_ref[...] = s
    b2_ref[...] = BETA * s * s
    bg_ref[...] = (BETA / GAMMA) * s


def _tc_scales(deg_parts):
    return pl.pallas_call(
        _scales_body,
        out_shape=[jax.ShapeDtypeStruct((1, N), jnp.float32)] * 3,
    )(deg_parts)


_BR = 2000


def _rowscale_body(s_ref, h_ref, o_ref):
    o_ref[...] = s_ref[...] * h_ref[...]


def _tc_rowscale(s_col, h):
    return pl.pallas_call(
        _rowscale_body,
        grid=(N // _BR,),
        in_specs=[
            pl.BlockSpec((_BR, 1), lambda i: (i, 0)),
            pl.BlockSpec((_BR, D), lambda i: (i, 0)),
        ],
        out_specs=pl.BlockSpec((_BR, D), lambda i: (i, 0)),
        out_shape=jax.ShapeDtypeStruct((N, D), jnp.float32),
    )(s_col, h)


def _combine_body(addmul, acc_ref, sc_ref, add_ref, o_ref):
    o_ref[...] = sc_ref[...] * acc_ref[...] + addmul * add_ref[...]


def _tc_combine(acc_flat, scale_col, addsrc, addmul):
    return pl.pallas_call(
        functools.partial(_combine_body, addmul),
        grid=(N // _BR,),
        in_specs=[
            pl.BlockSpec((_BR, D), lambda i: (i, 0)),
            pl.BlockSpec((_BR, 1), lambda i: (i, 0)),
            pl.BlockSpec((_BR, D), lambda i: (i, 0)),
        ],
        out_specs=pl.BlockSpec((_BR, D), lambda i: (i, 0)),
        out_shape=jax.ShapeDtypeStruct((N, D), jnp.float32),
    )(acc_flat, scale_col, addsrc)


def kernel(x, edge_index):
    row = edge_index[0]
    col = edge_index[1]

    brow, bcol, deg_parts = _sc_partition(row, col)
    browr = brow.reshape(NC, NS, NBB, BB)
    bcolr = bcol.reshape(NC, NS, NBB, BB)

    s_row, b2_row, bg_row = _tc_scales(deg_parts)
    s_col = s_row.reshape(N, 1)
    b2 = b2_row.reshape(N, 1)
    bg = bg_row.reshape(N, 1)

    ot = _tc_rowscale(s_col, x)
    ot0 = ot
    for k in range(K):
        acc = _sc_spmm(ot, browr, bcolr)
        acc_flat = acc.reshape(NC * LB, D)
        if k < K - 1:
            ot = _tc_combine(acc_flat, b2, ot0, ALPHA)
        else:
            out = _tc_combine(acc_flat, bg, x, ALPHA / GAMMA)
    return out
